# baseline (speedup 1.0000x reference)
import jax
import jax.numpy as jnp
from jax import lax
from jax.experimental import pallas as pl
from jax.experimental.pallas import tpu as pltpu
from jax.experimental.pallas import tpu_sc as plsc

N = 10000
C = 128
E = 320000
K = 10
ALPHA = 0.1
NC = 2
NS = 16
NW = NC * NS
STRIPE = 632
N_TAB = NS * STRIPE
CHUNK = 128
CPT = 80
HALF = CPT // 2
E_PAD = NW * CPT * CHUNK

_f32 = jnp.float32


def _sc_segsum_body(init0, init1, tab, pidx_t, es,
                    pidx_v, src_b, dst_b, brows_v, frows_v, acc_s, *gsems):
  cid = lax.axis_index("c")
  tid = lax.axis_index("s")
  wid = cid * NS + tid
  stripe = pl.ds(tid * STRIPE, STRIPE)

  for core, init in enumerate((init0, init1)):

    @pl.when(cid == core)
    def _():
      pltpu.sync_copy(init.at[stripe], acc_s.at[stripe])

  plsc.subcore_barrier()

  def unpack(lj, slot):
    base = lj * CHUNK
    for l in range(CHUNK // 16):
      v = pidx_v[pl.ds(base + l * 16, 16)]
      src_b[slot, pl.ds(l * 16, 16)] = v & 0xFFFF
      dst_b[slot, pl.ds(l * 16, 16)] = v >> 16

  def fire(slot):
    pltpu.async_copy(tab.at[src_b.at[slot]], brows_v.at[slot], gsems[slot])

  def gwait(slot):
    pltpu.make_async_copy(
        tab.at[src_b.at[slot]], brows_v.at[slot], gsems[slot]).wait()

  def convert(slot):
    def row_body(r4, carry):
      for rr in range(4):
        r = r4 * 4 + rr
        for g in range(C // 32):
          w = brows_v[slot, r, pl.ds(g * 16, 16)]
          lo = plsc.bitcast(w << 16, jnp.float32)
          hi = plsc.bitcast(w & jnp.int32(-65536), jnp.float32)
          frows_v[r, pl.ds(g * 32, 16)] = lo
          frows_v[r, pl.ds(g * 32 + 16, 16)] = hi
      return carry

    lax.fori_loop(0, CHUNK // 4, row_body, 0)

  for half in range(2):
    pltpu.sync_copy(
        pidx_t.at[wid, pl.ds(half * HALF * CHUNK, HALF * CHUNK)], pidx_v)
    unpack(0, 0)
    fire(0)

    def pair_body(p, carry):
      for b in range(2):
        lj = p * 2 + b

        @pl.when(lj + 1 < HALF)
        def _():
          unpack(lj + 1, 1 - b)
          fire(1 - b)

        gwait(b)
        convert(b)
        pltpu.sync_copy(frows_v, acc_s.at[dst_b.at[b]], add=True)
      return carry

    lax.fori_loop(0, HALF // 2, pair_body, 0)

  plsc.subcore_barrier()
  pltpu.sync_copy(acc_s.at[stripe], es.at[cid, stripe])


_sc_segsum = pl.kernel(
    _sc_segsum_body,
    out_type=jax.ShapeDtypeStruct((NC, N_TAB, C), _f32),
    mesh=plsc.VectorSubcoreMesh(
        core_axis_name="c", subcore_axis_name="s", num_cores=NC, num_subcores=NS
    ),
    scratch_types=[
        pltpu.VMEM((HALF * CHUNK,), jnp.int32),
        pltpu.VMEM((2, CHUNK), jnp.int32),
        pltpu.VMEM((2, CHUNK), jnp.int32),
        pltpu.VMEM((2, CHUNK, C // 2), jnp.int32),
        pltpu.VMEM((CHUNK, C), _f32),
        pltpu.VMEM_SHARED((N_TAB, C), _f32),
    ] + [pltpu.SemaphoreType.DMA] * 2,
    compiler_params=pltpu.CompilerParams(use_tc_tiling_on_sc=False, needs_layout_passes=False),
)


def _prep_body(x_ref, deg_ref, dinv_ref, zn_ref):
  deg = deg_ref[0, 0:N, :] + deg_ref[1, 0:N, :]
  dinv = lax.rsqrt(deg)
  dinv_ref[...] = dinv
  zn_ref[0:N, :] = x_ref[...] * dinv
  zn_ref[N:N_TAB, :] = jnp.zeros((N_TAB - N, C), _f32)


_prep = pl.pallas_call(
    _prep_body,
    out_shape=(
        jax.ShapeDtypeStruct((N, C), _f32),
        jax.ShapeDtypeStruct((N_TAB, C), _f32),
    ),
)


def _step_body(x_ref, dinv_ref, es_ref, zn_ref):
  es = es_ref[0, 0:N, :] + es_ref[1, 0:N, :]
  dinv = dinv_ref[...]
  z_new = (1.0 - ALPHA) * dinv * es + ALPHA * x_ref[...]
  zn_ref[0:N, :] = z_new * dinv
  zn_ref[N:N_TAB, :] = jnp.zeros((N_TAB - N, C), _f32)


_step = pl.pallas_call(
    _step_body,
    out_shape=jax.ShapeDtypeStruct((N_TAB, C), _f32),
)


def _final_body(x_ref, x_pre_ref, dinv_ref, es_ref, out_ref):
  es = es_ref[0, 0:N, :] + es_ref[1, 0:N, :]
  x = x_ref[...]
  z_k = (1.0 - ALPHA) * dinv_ref[...] * es + ALPHA * x
  out_ref[...] = 2.0 * x + z_k - x_pre_ref[...]


_final = pl.pallas_call(
    _final_body,
    out_shape=jax.ShapeDtypeStruct((N, C), _f32),
)


def kernel(x, x_pre, edge_index):
  src = edge_index[0]
  dst = edge_index[1]
  pad = E_PAD - E
  padv = (N + jnp.arange(pad, dtype=jnp.int32) % (N_TAB - N)).astype(jnp.int32)
  srcp = jnp.concatenate([src, padv])
  dstp = jnp.concatenate([dst, padv])
  pidx_t = (srcp | (dstp << 16)).reshape(NW, CPT * CHUNK)

  def to_tab(zn):
    zb = zn.astype(jnp.bfloat16).reshape(N_TAB, C // 32, 2, 16)
    st = jnp.stack([zb[:, :, 0, :], zb[:, :, 1, :]], axis=-1)
    return jax.lax.bitcast_convert_type(st, jnp.int32).reshape(N_TAB, C // 2)

  ones_tab = jnp.concatenate(
      [jnp.ones((N, C), _f32), jnp.zeros((N_TAB - N, C), _f32)]
  )
  zeros_tab = jnp.zeros((N_TAB, C), _f32)

  es_deg = _sc_segsum(ones_tab, zeros_tab, to_tab(ones_tab), pidx_t)
  dinv, zn = _prep(x, es_deg)

  for _ in range(K - 1):
    es = _sc_segsum(zn, zeros_tab, to_tab(zn), pidx_t)
    zn = _step(x, dinv, es)

  es = _sc_segsum(zn, zeros_tab, to_tab(zn), pidx_t)
  return _final(x, x_pre, dinv, es)


# speedup vs baseline: 2.3396x; 2.3396x over previous
import jax
import jax.numpy as jnp
from jax import lax
from jax.experimental import pallas as pl
from jax.experimental.pallas import tpu as pltpu
from jax.experimental.pallas import tpu_sc as plsc

N = 10000
C = 128
E = 320000
K = 10
ALPHA = 0.1
NC = 2
NS = 16
NW = NC * NS
STRIPE = 632
N_TAB = NS * STRIPE
CHUNK = 128
CPT = 80
HALF = CPT // 2
E_PAD = NW * CPT * CHUNK

_f32 = jnp.float32


def _sc_segsum_body(init0, init1, tab, pidx_t, es,
                    pidx_v, src_b, dst_b, rows_v, acc_s, *sems):
  gsems = sems[:2]
  ssems = sems[2:]
  cid = lax.axis_index("c")
  tid = lax.axis_index("s")
  wid = cid * NS + tid
  stripe = pl.ds(tid * STRIPE, STRIPE)

  for core, init in enumerate((init0, init1)):

    @pl.when(cid == core)
    def _():
      pltpu.sync_copy(init.at[stripe], acc_s.at[stripe])

  plsc.subcore_barrier()

  def unpack(lj, slot):
    base = lj * CHUNK
    for l in range(CHUNK // 16):
      v = pidx_v[pl.ds(base + l * 16, 16)]
      src_b[slot, pl.ds(l * 16, 16)] = v & 0xFFFF
      dst_b[slot, pl.ds(l * 16, 16)] = v >> 16

  def fire(slot):
    pltpu.async_copy(tab.at[src_b.at[slot]], rows_v.at[slot], gsems[slot])

  def gwait(slot):
    pltpu.make_async_copy(
        tab.at[src_b.at[slot]], rows_v.at[slot], gsems[slot]).wait()

  for half in range(2):
    pltpu.sync_copy(
        pidx_t.at[wid, pl.ds(half * HALF * CHUNK, HALF * CHUNK)], pidx_v)
    unpack(0, 0)
    fire(0)

    def pair_body(p, carry):
      for b in range(2):
        lj = p * 2 + b

        def swait():
          pltpu.make_async_copy(
              rows_v.at[1 - b], acc_s.at[dst_b.at[1 - b]], ssems[1 - b]).wait()

        if b == 1:
          swait()
        else:
          pl.when(p > 0)(swait)

        @pl.when(lj + 1 < HALF)
        def _():
          unpack(lj + 1, 1 - b)
          fire(1 - b)

        gwait(b)
        pltpu.async_copy(rows_v.at[b], acc_s.at[dst_b.at[b]], ssems[b],
                         add=True)
      return carry

    lax.fori_loop(0, HALF // 2, pair_body, 0)

    pltpu.make_async_copy(
        rows_v.at[1], acc_s.at[dst_b.at[1]], ssems[1]).wait()

  plsc.subcore_barrier()
  pltpu.sync_copy(acc_s.at[stripe], es.at[cid, stripe])


_sc_segsum = pl.kernel(
    _sc_segsum_body,
    out_type=jax.ShapeDtypeStruct((NC, N_TAB, C), _f32),
    mesh=plsc.VectorSubcoreMesh(
        core_axis_name="c", subcore_axis_name="s", num_cores=NC, num_subcores=NS
    ),
    scratch_types=[
        pltpu.VMEM((HALF * CHUNK,), jnp.int32),
        pltpu.VMEM((2, CHUNK), jnp.int32),
        pltpu.VMEM((2, CHUNK), jnp.int32),
        pltpu.VMEM((2, CHUNK, C), _f32),
        pltpu.VMEM_SHARED((N_TAB, C), _f32),
    ] + [pltpu.SemaphoreType.DMA] * 4,
)


def _prep_body(x_ref, deg_ref, dinv_ref, zn_ref):
  deg = deg_ref[0, 0:N, :] + deg_ref[1, 0:N, :]
  dinv = lax.rsqrt(deg)
  dinv_ref[...] = dinv
  zn_ref[0:N, :] = x_ref[...] * dinv
  zn_ref[N:N_TAB, :] = jnp.zeros((N_TAB - N, C), _f32)


_prep = pl.pallas_call(
    _prep_body,
    out_shape=(
        jax.ShapeDtypeStruct((N, C), _f32),
        jax.ShapeDtypeStruct((N_TAB, C), _f32),
    ),
)


def _step_body(x_ref, dinv_ref, es_ref, zn_ref):
  es = es_ref[0, 0:N, :] + es_ref[1, 0:N, :]
  dinv = dinv_ref[...]
  z_new = (1.0 - ALPHA) * dinv * es + ALPHA * x_ref[...]
  zn_ref[0:N, :] = z_new * dinv
  zn_ref[N:N_TAB, :] = jnp.zeros((N_TAB - N, C), _f32)


_step = pl.pallas_call(
    _step_body,
    out_shape=jax.ShapeDtypeStruct((N_TAB, C), _f32),
)


def _final_body(x_ref, x_pre_ref, dinv_ref, es_ref, out_ref):
  es = es_ref[0, 0:N, :] + es_ref[1, 0:N, :]
  x = x_ref[...]
  z_k = (1.0 - ALPHA) * dinv_ref[...] * es + ALPHA * x
  out_ref[...] = 2.0 * x + z_k - x_pre_ref[...]


_final = pl.pallas_call(
    _final_body,
    out_shape=jax.ShapeDtypeStruct((N, C), _f32),
)


def kernel(x, x_pre, edge_index):
  src = edge_index[0]
  dst = edge_index[1]
  pad = E_PAD - E
  padv = (N + jnp.arange(pad, dtype=jnp.int32) % (N_TAB - N)).astype(jnp.int32)
  srcp = jnp.concatenate([src, padv])
  dstp = jnp.concatenate([dst, padv])
  pidx_t = (srcp | (dstp << 16)).reshape(NW, CPT * CHUNK)

  ones_tab = jnp.concatenate(
      [jnp.ones((N, C), _f32), jnp.zeros((N_TAB - N, C), _f32)]
  )
  zeros_tab = jnp.zeros((N_TAB, C), _f32)

  es_deg = _sc_segsum(ones_tab, zeros_tab, ones_tab, pidx_t)
  dinv, zn = _prep(x, es_deg)

  for _ in range(K - 1):
    es = _sc_segsum(zn, zeros_tab, zn, pidx_t)
    zn = _step(x, dinv, es)

  es = _sc_segsum(zn, zeros_tab, zn, pidx_t)
  return _final(x, x_pre, dinv, es)
